# Initial kernel scaffold; baseline (speedup 1.0000x reference)
import jax
import jax.numpy as jnp
from jax.experimental import pallas as pl


def kernel(feat, edge_index, params):
    raise NotImplementedError("write your pallas kernel here")


# speedup vs baseline: 1.0599x; 1.0599x over previous
import functools

import jax
import jax.numpy as jnp
from jax.experimental import pallas as pl
from jax.experimental.pallas import tpu as pltpu

BN_EPS = 1e-5
NEG_BIG = -1e9


def _lstm_step(x, h, c, wih, whh, bias):
    g = (jnp.dot(x, wih, preferred_element_type=jnp.float32)
         + jnp.dot(h, whh, preferred_element_type=jnp.float32) + bias)
    H = h.shape[-1]
    gi = g[:, 0:H]
    gf = g[:, H:2 * H]
    gg = g[:, 2 * H:3 * H]
    go = g[:, 3 * H:4 * H]
    c2 = jax.nn.sigmoid(gf) * c + jax.nn.sigmoid(gi) * jnp.tanh(gg)
    h2 = jax.nn.sigmoid(go) * jnp.tanh(c2)
    return h2, c2


def _enc_body(Bblk, M, K, N, EMB, H,
              feat_ref, w1t_ref, b1_ref, w2t_ref, b2_ref,
              wih_ref, whh_ref, bias_ref,
              r_ref, part_ref, h_ref, c_ref):
    t = pl.program_id(1)

    @pl.when(t == 0)
    def _init():
        h_ref[...] = jnp.zeros_like(h_ref)
        c_ref[...] = jnp.zeros_like(c_ref)

    x5 = feat_ref[...].reshape(Bblk, M, K, 3)
    vis = x5[..., 2]
    w = (vis > 0.5).astype(jnp.float32)
    denom = jnp.maximum(w.sum(axis=-1), 1.0)
    rootx = (x5[..., 0] * w).sum(axis=-1) / denom
    rooty = (x5[..., 1] * w).sum(axis=-1) / denom
    rootv = (vis.max(axis=-1) > 0.5).astype(jnp.float32)
    root = jnp.stack([rootx, rooty, rootv], axis=-1)
    xall = jnp.concatenate([x5, root[:, :, None, :]], axis=2)
    X = xall.reshape(Bblk * N, 3)

    a1 = jax.nn.relu(jnp.dot(X, w1t_ref[...],
                             preferred_element_type=jnp.float32) + b1_ref[...])
    v = jnp.dot(a1, w2t_ref[...], preferred_element_type=jnp.float32) + b2_ref[...]

    h2, c2 = _lstm_step(v, h_ref[...], c_ref[...],
                        wih_ref[...], whh_ref[...], bias_ref[...])
    h_ref[...] = h2
    c_ref[...] = c2
    r_ref[...] = h2.reshape(Bblk, 1, N, H)
    s = h2.sum(axis=0)[None, :]
    sq = (h2 * h2).sum(axis=0)[None, :]
    part_ref[...] = jnp.concatenate([s, sq], axis=0).reshape(1, 1, 2, H)


def _bn_body(count, gamma_ref, beta_ref, part_ref, out_ref):
    p = part_ref[...].sum(axis=0)
    mean = p[:, 0, :] / count
    msq = p[:, 1, :] / count
    var = msq - mean * mean
    invstd = jax.lax.rsqrt(var + BN_EPS)
    scale = gamma_ref[...] * invstd
    shift = beta_ref[...] - mean * scale
    out_ref[...] = jnp.stack([scale, shift], axis=1)


def _gat(x, amask, wg, asr, ads, bg, Bblk, N, H):
    h1 = jnp.dot(x.reshape(Bblk * N, H), wg,
                 preferred_element_type=jnp.float32).reshape(Bblk, N, H)
    d = (h1 * ads).sum(axis=-1, keepdims=True)
    s = (h1 * asr).sum(axis=-1, keepdims=True)
    e = d + jnp.swapaxes(s, 1, 2)
    e = jnp.where(e >= 0, e, 0.2 * e)
    e = e + amask
    emax = e.max(axis=-1, keepdims=True)
    ex = jnp.exp(e - emax)
    den = ex.sum(axis=-1, keepdims=True)
    alpha = ex / jnp.maximum(den, 1e-16)
    outs = [jnp.dot(alpha[b], h1[b], preferred_element_type=jnp.float32)
            for b in range(Bblk)]
    out = jnp.stack(outs, axis=0) + bg
    return jnp.where(out > 0, out, jnp.exp(jnp.minimum(out, 0.0)) - 1.0)


def _dec_body(Bblk, N, H,
              r_ref, ss_ref, amask_ref,
              w1g_ref, a1s_ref, a1d_ref, b1g_ref,
              w2g_ref, a2s_ref, a2d_ref, b2g_ref,
              wih_ref, whh_ref, bias_ref,
              l_ref, h_ref, c_ref):
    t = pl.program_id(1)

    @pl.when(t == 0)
    def _init():
        h_ref[...] = jnp.zeros_like(h_ref)
        c_ref[...] = jnp.zeros_like(c_ref)

    ss = ss_ref[...]
    scale = ss[0, 0, :]
    shift = ss[0, 1, :]
    x = r_ref[...].reshape(Bblk, N, H) * scale + shift
    amask = amask_ref[...]
    x = _gat(x, amask, w1g_ref[...], a1s_ref[...], a1d_ref[...], b1g_ref[...],
             Bblk, N, H)
    x = _gat(x, amask, w2g_ref[...], a2s_ref[...], a2d_ref[...], b2g_ref[...],
             Bblk, N, H)
    h2, c2 = _lstm_step(x.reshape(Bblk * N, H), h_ref[...], c_ref[...],
                        wih_ref[...], whh_ref[...], bias_ref[...])
    h_ref[...] = h2
    c_ref[...] = c2
    l_ref[...] = h2.reshape(Bblk, 1, N, H)


def kernel(feat, edge_index, params):
    p = params
    B, T, M, K, _ = feat.shape
    KR = K + 1
    N = M * KR
    EMB = p['phi_w2'].shape[0]
    H = p['Whh_s'].shape[1]
    Bblk = 8
    G = B // Bblk
    f32 = jnp.float32

    w1t = p['phi_w1'].T
    b1 = p['phi_b1'][None, :]
    w2t = p['phi_w2'].T
    b2 = p['phi_b2'][None, :]
    wih_s = p['Wih_s'].T
    whh_s = p['Whh_s'].T
    bias_s = (p['bih_s'] + p['bhh_s'])[None, :]
    wih_t = p['Wih_t'].T
    whh_t = p['Whh_t'].T
    bias_t = (p['bih_t'] + p['bhh_t'])[None, :]
    w1g = p['g1_W'].T
    a1s = p['g1_asrc'][None, :]
    a1d = p['g1_adst'][None, :]
    b1g = p['g1_b'][None, :]
    w2g = p['g2_W'].T
    a2s = p['g2_asrc'][None, :]
    a2d = p['g2_adst'][None, :]
    b2g = p['g2_b'][None, :]
    src = edge_index[0]
    dst = edge_index[1]
    adj = jnp.zeros((N, N), f32).at[dst, src].add(1.0)
    amask = (1.0 - jnp.minimum(adj, 1.0)) * NEG_BIG

    const = lambda shape: pl.BlockSpec(shape, lambda i, t: tuple(0 for _ in shape))

    r, part = pl.pallas_call(
        functools.partial(_enc_body, Bblk, M, K, N, EMB, H),
        grid=(G, T),
        in_specs=[
            pl.BlockSpec((Bblk, 1, M, K, 3), lambda i, t: (i, t, 0, 0, 0)),
            const((3, EMB)), const((1, EMB)),
            const((EMB, EMB)), const((1, EMB)),
            const((EMB, 4 * H)), const((H, 4 * H)), const((1, 4 * H)),
        ],
        out_specs=[
            pl.BlockSpec((Bblk, 1, N, H), lambda i, t: (i, t, 0, 0)),
            pl.BlockSpec((1, 1, 2, H), lambda i, t: (i, t, 0, 0)),
        ],
        out_shape=[
            jax.ShapeDtypeStruct((B, T, N, H), f32),
            jax.ShapeDtypeStruct((G, T, 2, H), f32),
        ],
        scratch_shapes=[pltpu.VMEM((Bblk * N, H), f32),
                        pltpu.VMEM((Bblk * N, H), f32)],
        compiler_params=pltpu.CompilerParams(
            dimension_semantics=("parallel", "arbitrary")),
    )(feat, w1t, b1, w2t, b2, wih_s, whh_s, bias_s)

    ss = pl.pallas_call(
        functools.partial(_bn_body, float(B * N)),
        out_shape=jax.ShapeDtypeStruct((T, 2, H), f32),
    )(p['bn_gamma'][None, :], p['bn_beta'][None, :], part)

    l = pl.pallas_call(
        functools.partial(_dec_body, Bblk, N, H),
        grid=(G, T),
        in_specs=[
            pl.BlockSpec((Bblk, 1, N, H), lambda i, t: (i, t, 0, 0)),
            pl.BlockSpec((1, 2, H), lambda i, t: (t, 0, 0)),
            const((N, N)),
            const((H, H)), const((1, H)), const((1, H)), const((1, H)),
            const((H, H)), const((1, H)), const((1, H)), const((1, H)),
            const((H, 4 * H)), const((H, 4 * H)), const((1, 4 * H)),
        ],
        out_specs=pl.BlockSpec((Bblk, 1, N, H), lambda i, t: (i, t, 0, 0)),
        out_shape=jax.ShapeDtypeStruct((B, T, N, H), f32),
        scratch_shapes=[pltpu.VMEM((Bblk * N, H), f32),
                        pltpu.VMEM((Bblk * N, H), f32)],
        compiler_params=pltpu.CompilerParams(
            dimension_semantics=("parallel", "arbitrary")),
    )(r, ss, amask,
      w1g, a1s, a1d, b1g, w2g, a2s, a2d, b2g,
      wih_t, whh_t, bias_t)

    h = jnp.concatenate([r, l], axis=-1)
    return (r, h)


# speedup vs baseline: 1.7819x; 1.6812x over previous
import functools

import jax
import jax.numpy as jnp
from jax.experimental import pallas as pl
from jax.experimental.pallas import tpu as pltpu

BN_EPS = 1e-5
NEG_BIG = -1e9


def _lstm_step(x, h, c, wih, whh, bias):
    g = (jnp.dot(x, wih, preferred_element_type=jnp.float32)
         + jnp.dot(h, whh, preferred_element_type=jnp.float32) + bias)
    H = h.shape[-1]
    gi = g[:, 0:H]
    gf = g[:, H:2 * H]
    gg = g[:, 2 * H:3 * H]
    go = g[:, 3 * H:4 * H]
    c2 = jax.nn.sigmoid(gf) * c + jax.nn.sigmoid(gi) * jnp.tanh(gg)
    h2 = jax.nn.sigmoid(go) * jnp.tanh(c2)
    return h2, c2


def _enc_body(Bblk, M, K, N, EMB, H,
              feat_ref, w1t_ref, b1_ref, w2t_ref, b2_ref,
              wih_ref, whh_ref, bias_ref,
              r_ref, part_ref, h_ref, c_ref):
    t = pl.program_id(1)

    @pl.when(t == 0)
    def _init():
        h_ref[...] = jnp.zeros_like(h_ref)
        c_ref[...] = jnp.zeros_like(c_ref)

    x5 = feat_ref[...].reshape(Bblk, M, K, 3)
    vis = x5[..., 2]
    w = (vis > 0.5).astype(jnp.float32)
    denom = jnp.maximum(w.sum(axis=-1), 1.0)
    rootx = (x5[..., 0] * w).sum(axis=-1) / denom
    rooty = (x5[..., 1] * w).sum(axis=-1) / denom
    rootv = (vis.max(axis=-1) > 0.5).astype(jnp.float32)
    root = jnp.stack([rootx, rooty, rootv], axis=-1)
    xall = jnp.concatenate([x5, root[:, :, None, :]], axis=2)
    X = xall.reshape(Bblk * N, 3)

    a1 = jax.nn.relu(jnp.dot(X, w1t_ref[...],
                             preferred_element_type=jnp.float32) + b1_ref[...])
    v = jnp.dot(a1, w2t_ref[...], preferred_element_type=jnp.float32) + b2_ref[...]

    h2, c2 = _lstm_step(v, h_ref[...], c_ref[...],
                        wih_ref[...], whh_ref[...], bias_ref[...])
    h_ref[...] = h2
    c_ref[...] = c2
    r_ref[...] = h2.reshape(Bblk, 1, N, H)
    s = h2.sum(axis=0)[None, :]
    sq = (h2 * h2).sum(axis=0)[None, :]
    part_ref[...] = jnp.concatenate([s, sq], axis=0).reshape(1, 1, 2, H)


def _bn_body(count, gamma_ref, beta_ref, part_ref, out_ref):
    p = part_ref[...].sum(axis=0)
    mean = p[:, 0, :] / count
    msq = p[:, 1, :] / count
    var = msq - mean * mean
    invstd = jax.lax.rsqrt(var + BN_EPS)
    scale = gamma_ref[...] * invstd
    shift = beta_ref[...] - mean * scale
    out_ref[...] = jnp.stack([scale, shift], axis=1)


def _gat(x, amask, wg, asr, ads, bg, Bblk, N, H):
    h1 = jnp.dot(x.reshape(Bblk * N, H), wg,
                 preferred_element_type=jnp.float32).reshape(Bblk, N, H)
    d = (h1 * ads).sum(axis=-1, keepdims=True)
    s = (h1 * asr).sum(axis=-1, keepdims=True)
    e = d + jnp.swapaxes(s, 1, 2)
    e = jnp.where(e >= 0, e, 0.2 * e)
    e = e + amask
    emax = e.max(axis=-1, keepdims=True)
    ex = jnp.exp(e - emax)
    den = ex.sum(axis=-1, keepdims=True)
    alpha = ex / jnp.maximum(den, 1e-16)
    out = jax.lax.dot_general(
        alpha, h1, (((2,), (1,)), ((0,), (0,))),
        preferred_element_type=jnp.float32) + bg
    return jnp.where(out > 0, out, jnp.exp(jnp.minimum(out, 0.0)) - 1.0)


def _dec_body(Bblk, N, H,
              r_ref, ss_ref, amask_ref,
              w1g_ref, a1s_ref, a1d_ref, b1g_ref,
              w2g_ref, a2s_ref, a2d_ref, b2g_ref,
              wih_ref, whh_ref, bias_ref,
              l_ref, h_ref, c_ref):
    t = pl.program_id(1)

    @pl.when(t == 0)
    def _init():
        h_ref[...] = jnp.zeros_like(h_ref)
        c_ref[...] = jnp.zeros_like(c_ref)

    ss = ss_ref[...]
    scale = ss[0, 0, :]
    shift = ss[0, 1, :]
    x = r_ref[...].reshape(Bblk, N, H) * scale + shift
    amask = amask_ref[...]
    x = _gat(x, amask, w1g_ref[...], a1s_ref[...], a1d_ref[...], b1g_ref[...],
             Bblk, N, H)
    x = _gat(x, amask, w2g_ref[...], a2s_ref[...], a2d_ref[...], b2g_ref[...],
             Bblk, N, H)
    h2, c2 = _lstm_step(x.reshape(Bblk * N, H), h_ref[...], c_ref[...],
                        wih_ref[...], whh_ref[...], bias_ref[...])
    h_ref[...] = h2
    c_ref[...] = c2
    l_ref[...] = h2.reshape(Bblk, 1, N, H)


def kernel(feat, edge_index, params):
    p = params
    B, T, M, K, _ = feat.shape
    KR = K + 1
    N = M * KR
    EMB = p['phi_w2'].shape[0]
    H = p['Whh_s'].shape[1]
    Bblk = 32
    G = B // Bblk
    f32 = jnp.float32

    w1t = p['phi_w1'].T
    b1 = p['phi_b1'][None, :]
    w2t = p['phi_w2'].T
    b2 = p['phi_b2'][None, :]
    wih_s = p['Wih_s'].T
    whh_s = p['Whh_s'].T
    bias_s = (p['bih_s'] + p['bhh_s'])[None, :]
    wih_t = p['Wih_t'].T
    whh_t = p['Whh_t'].T
    bias_t = (p['bih_t'] + p['bhh_t'])[None, :]
    w1g = p['g1_W'].T
    a1s = p['g1_asrc'][None, :]
    a1d = p['g1_adst'][None, :]
    b1g = p['g1_b'][None, :]
    w2g = p['g2_W'].T
    a2s = p['g2_asrc'][None, :]
    a2d = p['g2_adst'][None, :]
    b2g = p['g2_b'][None, :]
    src = edge_index[0]
    dst = edge_index[1]
    adj = jnp.zeros((N, N), f32).at[dst, src].add(1.0)
    amask = (1.0 - jnp.minimum(adj, 1.0)) * NEG_BIG

    const = lambda shape: pl.BlockSpec(shape, lambda i, t: tuple(0 for _ in shape))

    r, part = pl.pallas_call(
        functools.partial(_enc_body, Bblk, M, K, N, EMB, H),
        grid=(G, T),
        in_specs=[
            pl.BlockSpec((Bblk, 1, M, K, 3), lambda i, t: (i, t, 0, 0, 0)),
            const((3, EMB)), const((1, EMB)),
            const((EMB, EMB)), const((1, EMB)),
            const((EMB, 4 * H)), const((H, 4 * H)), const((1, 4 * H)),
        ],
        out_specs=[
            pl.BlockSpec((Bblk, 1, N, H), lambda i, t: (i, t, 0, 0)),
            pl.BlockSpec((1, 1, 2, H), lambda i, t: (i, t, 0, 0)),
        ],
        out_shape=[
            jax.ShapeDtypeStruct((B, T, N, H), f32),
            jax.ShapeDtypeStruct((G, T, 2, H), f32),
        ],
        scratch_shapes=[pltpu.VMEM((Bblk * N, H), f32),
                        pltpu.VMEM((Bblk * N, H), f32)],
        compiler_params=pltpu.CompilerParams(
            dimension_semantics=("parallel", "arbitrary")),
    )(feat, w1t, b1, w2t, b2, wih_s, whh_s, bias_s)

    ss = pl.pallas_call(
        functools.partial(_bn_body, float(B * N)),
        out_shape=jax.ShapeDtypeStruct((T, 2, H), f32),
    )(p['bn_gamma'][None, :], p['bn_beta'][None, :], part)

    l = pl.pallas_call(
        functools.partial(_dec_body, Bblk, N, H),
        grid=(G, T),
        in_specs=[
            pl.BlockSpec((Bblk, 1, N, H), lambda i, t: (i, t, 0, 0)),
            pl.BlockSpec((1, 2, H), lambda i, t: (t, 0, 0)),
            const((N, N)),
            const((H, H)), const((1, H)), const((1, H)), const((1, H)),
            const((H, H)), const((1, H)), const((1, H)), const((1, H)),
            const((H, 4 * H)), const((H, 4 * H)), const((1, 4 * H)),
        ],
        out_specs=pl.BlockSpec((Bblk, 1, N, H), lambda i, t: (i, t, 0, 0)),
        out_shape=jax.ShapeDtypeStruct((B, T, N, H), f32),
        scratch_shapes=[pltpu.VMEM((Bblk * N, H), f32),
                        pltpu.VMEM((Bblk * N, H), f32)],
        compiler_params=pltpu.CompilerParams(
            dimension_semantics=("parallel", "arbitrary")),
    )(r, ss, amask,
      w1g, a1s, a1d, b1g, w2g, a2s, a2d, b2g,
      wih_t, whh_t, bias_t)

    h = jnp.concatenate([r, l], axis=-1)
    return (r, h)


# speedup vs baseline: 2.4405x; 1.3697x over previous
import functools

import jax
import jax.numpy as jnp
from jax.experimental import pallas as pl
from jax.experimental.pallas import tpu as pltpu

BN_EPS = 1e-5
NEG_BIG = -1e9


def _lstm_step(x, h, c, wih4_ref, whh4_ref, b4_ref):
    def gate(q):
        return (jnp.dot(x, wih4_ref[q], preferred_element_type=jnp.float32)
                + jnp.dot(h, whh4_ref[q], preferred_element_type=jnp.float32)
                + b4_ref[q])
    gi, gf, gg, go = gate(0), gate(1), gate(2), gate(3)
    c2 = jax.nn.sigmoid(gf) * c + jax.nn.sigmoid(gi) * jnp.tanh(gg)
    h2 = jax.nn.sigmoid(go) * jnp.tanh(c2)
    return h2, c2


def _enc_body(Bblk, MK, N, EMB, H,
              xx_ref, yy_ref, vv_ref, S_ref, PL_ref, PR_ref,
              w1t_ref, b1_ref, w2t_ref, b2_ref,
              wih4_ref, whh4_ref, b4_ref,
              r_ref, part_ref, h_ref, c_ref):
    t = pl.program_id(1)

    @pl.when(t == 0)
    def _init():
        h_ref[...] = jnp.zeros_like(h_ref)
        c_ref[...] = jnp.zeros_like(c_ref)

    xx = xx_ref[...].reshape(Bblk, MK)
    yy = yy_ref[...].reshape(Bblk, MK)
    vv = vv_ref[...].reshape(Bblk, MK)
    w = (vv > 0.5).astype(jnp.float32)
    S = S_ref[...]
    sumw = jnp.dot(w, S, preferred_element_type=jnp.float32)
    denom = jnp.maximum(sumw, 1.0)
    rootx = jnp.dot(xx * w, S, preferred_element_type=jnp.float32) / denom
    rooty = jnp.dot(yy * w, S, preferred_element_type=jnp.float32) / denom
    rootv = (sumw > 0.5).astype(jnp.float32)
    PL = PL_ref[...]
    PR = PR_ref[...]

    def nodes(leaf, root):
        return (jnp.dot(leaf, PL, preferred_element_type=jnp.float32)
                + jnp.dot(root, PR, preferred_element_type=jnp.float32))
    xN = nodes(xx, rootx)
    yN = nodes(yy, rooty)
    vN = nodes(vv, rootv)
    X = jnp.stack([xN, yN, vN], axis=-1).reshape(Bblk * N, 3)
    a1 = jax.nn.relu(jnp.dot(X, w1t_ref[...],
                             preferred_element_type=jnp.float32) + b1_ref[...])
    v = jnp.dot(a1, w2t_ref[...], preferred_element_type=jnp.float32) + b2_ref[...]

    h2, c2 = _lstm_step(v, h_ref[...], c_ref[...], wih4_ref, whh4_ref, b4_ref)
    h_ref[...] = h2
    c_ref[...] = c2
    r_ref[...] = h2.reshape(Bblk, 1, N, H)
    s = h2.sum(axis=0)[None, :]
    sq = (h2 * h2).sum(axis=0)[None, :]
    part_ref[...] = jnp.concatenate([s, sq], axis=0).reshape(1, 1, 2, H)


def _bn_body(count, gamma_ref, beta_ref, part_ref, out_ref):
    p = part_ref[...].sum(axis=0)
    mean = p[:, 0, :] / count
    msq = p[:, 1, :] / count
    var = msq - mean * mean
    invstd = jax.lax.rsqrt(var + BN_EPS)
    scale = gamma_ref[...] * invstd
    shift = beta_ref[...] - mean * scale
    out_ref[...] = jnp.stack([scale, shift], axis=1)


def _gat(x, amask, wg, asr, ads, bg, Bblk, N, H):
    h1 = jnp.dot(x.reshape(Bblk * N, H), wg,
                 preferred_element_type=jnp.float32).reshape(Bblk, N, H)
    d3 = jax.lax.dot_general(h1, ads, (((2,), (0,)), ((), ())),
                             preferred_element_type=jnp.float32)
    s3 = jax.lax.dot_general(h1, asr, (((2,), (0,)), ((), ())),
                             preferred_element_type=jnp.float32)
    ones = jnp.ones_like(d3)
    A = jnp.concatenate([d3, ones], axis=-1)
    Bm = jnp.concatenate([ones, s3], axis=-1)
    e = jax.lax.dot_general(A, Bm, (((2,), (2,)), ((0,), (0,))),
                            preferred_element_type=jnp.float32)
    e = jnp.where(e >= 0, e, 0.2 * e) + amask
    emax = e.max(axis=-1, keepdims=True)
    ex = jnp.exp(e - emax)
    den = ex.sum(axis=-1, keepdims=True)
    alpha = ex / jnp.maximum(den, 1e-16)
    out = jax.lax.dot_general(
        alpha, h1, (((2,), (1,)), ((0,), (0,))),
        preferred_element_type=jnp.float32) + bg
    return jnp.where(out > 0, out, jnp.exp(jnp.minimum(out, 0.0)) - 1.0)


def _dec_body(Bblk, N, H,
              r_ref, ss_ref, amask_ref,
              w1g_ref, a1s_ref, a1d_ref, b1g_ref,
              w2g_ref, a2s_ref, a2d_ref, b2g_ref,
              wih4_ref, whh4_ref, b4_ref,
              h_out_ref, h_ref, c_ref):
    t = pl.program_id(1)

    @pl.when(t == 0)
    def _init():
        h_ref[...] = jnp.zeros_like(h_ref)
        c_ref[...] = jnp.zeros_like(c_ref)

    ss = ss_ref[...]
    scale = ss[0, 0, :]
    shift = ss[0, 1, :]
    r_raw = r_ref[...].reshape(Bblk, N, H)
    x = r_raw * scale + shift
    amask = amask_ref[...]
    x = _gat(x, amask, w1g_ref[...], a1s_ref[...], a1d_ref[...], b1g_ref[...],
             Bblk, N, H)
    x = _gat(x, amask, w2g_ref[...], a2s_ref[...], a2d_ref[...], b2g_ref[...],
             Bblk, N, H)
    h2, c2 = _lstm_step(x.reshape(Bblk * N, H), h_ref[...], c_ref[...],
                        wih4_ref, whh4_ref, b4_ref)
    h_ref[...] = h2
    c_ref[...] = c2
    h_out_ref[...] = jnp.concatenate(
        [r_raw, h2.reshape(Bblk, N, H)], axis=-1).reshape(Bblk, 1, N, 2 * H)


def _gate_slabs(Wih, Whh, bih, bhh):
    H = Whh.shape[1]
    IN = Wih.shape[1]
    wih4 = Wih.reshape(4, H, IN).transpose(0, 2, 1)
    whh4 = Whh.reshape(4, H, H).transpose(0, 2, 1)
    b4 = (bih + bhh).reshape(4, 1, H)
    return wih4, whh4, b4


def kernel(feat, edge_index, params):
    p = params
    B, T, M, K, _ = feat.shape
    KR = K + 1
    MK = M * K
    N = M * KR
    EMB = p['phi_w2'].shape[0]
    H = p['Whh_s'].shape[1]
    Bblk = min(32, B)
    G = B // Bblk
    f32 = jnp.float32

    xx = feat[..., 0].reshape(B, T, 1, MK)
    yy = feat[..., 1].reshape(B, T, 1, MK)
    vv = feat[..., 2].reshape(B, T, 1, MK)
    w1t = p['phi_w1'].T
    b1 = p['phi_b1'][None, :]
    w2t = p['phi_w2'].T
    b2 = p['phi_b2'][None, :]
    wih4_s, whh4_s, b4_s = _gate_slabs(p['Wih_s'], p['Whh_s'],
                                       p['bih_s'], p['bhh_s'])
    wih4_t, whh4_t, b4_t = _gate_slabs(p['Wih_t'], p['Whh_t'],
                                       p['bih_t'], p['bhh_t'])
    w1g = p['g1_W'].T
    a1s = p['g1_asrc'][:, None]
    a1d = p['g1_adst'][:, None]
    b1g = p['g1_b'][None, :]
    w2g = p['g2_W'].T
    a2s = p['g2_asrc'][:, None]
    a2d = p['g2_adst'][:, None]
    b2g = p['g2_b'][None, :]

    leaf = jnp.arange(MK)
    S = (leaf[:, None] // K == jnp.arange(M)[None, :]).astype(f32)
    PL = (((leaf // K) * KR + leaf % K)[:, None]
          == jnp.arange(N)[None, :]).astype(f32)
    PR = (jnp.arange(M)[:, None] * KR + K
          == jnp.arange(N)[None, :]).astype(f32)

    src = edge_index[0]
    dst = edge_index[1]
    adj = jnp.zeros((N, N), f32).at[dst, src].add(1.0)
    amask = (1.0 - jnp.minimum(adj, 1.0)) * NEG_BIG

    const = lambda shape: pl.BlockSpec(shape, lambda i, t: tuple(0 for _ in shape))

    chan_spec = pl.BlockSpec((Bblk, 1, 1, MK), lambda i, t: (i, t, 0, 0))
    r, part = pl.pallas_call(
        functools.partial(_enc_body, Bblk, MK, N, EMB, H),
        grid=(G, T),
        in_specs=[
            chan_spec, chan_spec, chan_spec,
            const((MK, M)), const((MK, N)), const((M, N)),
            const((3, EMB)), const((1, EMB)),
            const((EMB, EMB)), const((1, EMB)),
            const((4, EMB, H)), const((4, H, H)), const((4, 1, H)),
        ],
        out_specs=[
            pl.BlockSpec((Bblk, 1, N, H), lambda i, t: (i, t, 0, 0)),
            pl.BlockSpec((1, 1, 2, H), lambda i, t: (i, t, 0, 0)),
        ],
        out_shape=[
            jax.ShapeDtypeStruct((B, T, N, H), f32),
            jax.ShapeDtypeStruct((G, T, 2, H), f32),
        ],
        scratch_shapes=[pltpu.VMEM((Bblk * N, H), f32),
                        pltpu.VMEM((Bblk * N, H), f32)],
        compiler_params=pltpu.CompilerParams(
            dimension_semantics=("parallel", "arbitrary")),
    )(xx, yy, vv, S, PL, PR, w1t, b1, w2t, b2, wih4_s, whh4_s, b4_s)

    ss = pl.pallas_call(
        functools.partial(_bn_body, float(B * N)),
        out_shape=jax.ShapeDtypeStruct((T, 2, H), f32),
    )(p['bn_gamma'][None, :], p['bn_beta'][None, :], part)

    h = pl.pallas_call(
        functools.partial(_dec_body, Bblk, N, H),
        grid=(G, T),
        in_specs=[
            pl.BlockSpec((Bblk, 1, N, H), lambda i, t: (i, t, 0, 0)),
            pl.BlockSpec((1, 2, H), lambda i, t: (t, 0, 0)),
            const((N, N)),
            const((H, H)), const((H, 1)), const((H, 1)), const((1, H)),
            const((H, H)), const((H, 1)), const((H, 1)), const((1, H)),
            const((4, H, H)), const((4, H, H)), const((4, 1, H)),
        ],
        out_specs=pl.BlockSpec((Bblk, 1, N, 2 * H), lambda i, t: (i, t, 0, 0)),
        out_shape=jax.ShapeDtypeStruct((B, T, N, 2 * H), f32),
        scratch_shapes=[pltpu.VMEM((Bblk * N, H), f32),
                        pltpu.VMEM((Bblk * N, H), f32)],
        compiler_params=pltpu.CompilerParams(
            dimension_semantics=("parallel", "arbitrary")),
    )(r, ss, amask,
      w1g, a1s, a1d, b1g, w2g, a2s, a2d, b2g,
      wih4_t, whh4_t, b4_t)

    return (r, h)


# speedup vs baseline: 2.7244x; 1.1163x over previous
import functools

import jax
import jax.numpy as jnp
from jax.experimental import pallas as pl
from jax.experimental.pallas import tpu as pltpu

BN_EPS = 1e-5
NEG_BIG = -1e9


def _lstm_step(x, h, c, wih4_ref, whh4_ref, b4_ref):
    def gate(q):
        return (jnp.dot(x, wih4_ref[q], preferred_element_type=jnp.float32)
                + jnp.dot(h, whh4_ref[q], preferred_element_type=jnp.float32)
                + b4_ref[q])
    gi, gf, gg, go = gate(0), gate(1), gate(2), gate(3)
    c2 = jax.nn.sigmoid(gf) * c + jax.nn.sigmoid(gi) * jnp.tanh(gg)
    h2 = jax.nn.sigmoid(go) * jnp.tanh(c2)
    return h2, c2


def _enc_body(Bblk, TB, MK, N, EMB, H,
              xx_ref, yy_ref, vv_ref, S_ref, PL_ref, PR_ref,
              w1t_ref, b1_ref, w2t_ref, b2_ref,
              wih4_ref, whh4_ref, b4_ref,
              r_ref, part_ref, h_ref, c_ref):
    t = pl.program_id(1)

    @pl.when(t == 0)
    def _init():
        h_ref[...] = jnp.zeros_like(h_ref)
        c_ref[...] = jnp.zeros_like(c_ref)

    S = S_ref[...]
    PL = PL_ref[...]
    PR = PR_ref[...]
    for u in range(TB):
        xx = xx_ref[:, u].reshape(Bblk, MK)
        yy = yy_ref[:, u].reshape(Bblk, MK)
        vv = vv_ref[:, u].reshape(Bblk, MK)
        w = (vv > 0.5).astype(jnp.float32)
        sumw = jnp.dot(w, S, preferred_element_type=jnp.float32)
        denom = jnp.maximum(sumw, 1.0)
        rootx = jnp.dot(xx * w, S, preferred_element_type=jnp.float32) / denom
        rooty = jnp.dot(yy * w, S, preferred_element_type=jnp.float32) / denom
        rootv = (sumw > 0.5).astype(jnp.float32)

        def nodes(leaf, root):
            return (jnp.dot(leaf, PL, preferred_element_type=jnp.float32)
                    + jnp.dot(root, PR, preferred_element_type=jnp.float32))
        xN = nodes(xx, rootx)
        yN = nodes(yy, rooty)
        vN = nodes(vv, rootv)
        X = jnp.stack([xN, yN, vN], axis=-1).reshape(Bblk * N, 3)
        a1 = jax.nn.relu(jnp.dot(X, w1t_ref[...],
                                 preferred_element_type=jnp.float32) + b1_ref[...])
        v = (jnp.dot(a1, w2t_ref[...], preferred_element_type=jnp.float32)
             + b2_ref[...])

        h2, c2 = _lstm_step(v, h_ref[...], c_ref[...],
                            wih4_ref, whh4_ref, b4_ref)
        h_ref[...] = h2
        c_ref[...] = c2
        r_ref[:, u] = h2.reshape(Bblk, N, H)
        s = h2.sum(axis=0)[None, :]
        sq = (h2 * h2).sum(axis=0)[None, :]
        part_ref[:, u] = jnp.concatenate([s, sq], axis=0).reshape(1, 2, H)


def _bn_body(count, gamma_ref, beta_ref, part_ref, out_ref):
    p = part_ref[...].sum(axis=0)
    mean = p[:, 0, :] / count
    msq = p[:, 1, :] / count
    var = msq - mean * mean
    invstd = jax.lax.rsqrt(var + BN_EPS)
    scale = gamma_ref[...] * invstd
    shift = beta_ref[...] - mean * scale
    out_ref[...] = jnp.stack([scale, shift], axis=1)


def _gat(x, amask, wg, asr, ads, bg, Bblk, N, H):
    h1 = jnp.dot(x.reshape(Bblk * N, H), wg,
                 preferred_element_type=jnp.float32).reshape(Bblk, N, H)
    d3 = jax.lax.dot_general(h1, ads, (((2,), (0,)), ((), ())),
                             preferred_element_type=jnp.float32)
    s3 = jax.lax.dot_general(h1, asr, (((2,), (0,)), ((), ())),
                             preferred_element_type=jnp.float32)
    ones = jnp.ones_like(d3)
    A = jnp.concatenate([d3, ones], axis=-1)
    Bm = jnp.concatenate([ones, s3], axis=-1)
    e = jax.lax.dot_general(A, Bm, (((2,), (2,)), ((0,), (0,))),
                            preferred_element_type=jnp.float32)
    e = jnp.where(e >= 0, e, 0.2 * e) + amask
    emax = e.max(axis=-1, keepdims=True)
    ex = jnp.exp(e - emax)
    den = ex.sum(axis=-1, keepdims=True)
    alpha = ex / jnp.maximum(den, 1e-16)
    out = jax.lax.dot_general(
        alpha, h1, (((2,), (1,)), ((0,), (0,))),
        preferred_element_type=jnp.float32) + bg
    return jnp.where(out > 0, out, jnp.exp(jnp.minimum(out, 0.0)) - 1.0)


def _dec_body(Bblk, TB, N, H,
              r_ref, ss_ref, amask_ref,
              w1g_ref, a1s_ref, a1d_ref, b1g_ref,
              w2g_ref, a2s_ref, a2d_ref, b2g_ref,
              wih4_ref, whh4_ref, b4_ref,
              h_out_ref, h_ref, c_ref):
    t = pl.program_id(1)

    @pl.when(t == 0)
    def _init():
        h_ref[...] = jnp.zeros_like(h_ref)
        c_ref[...] = jnp.zeros_like(c_ref)

    amask = amask_ref[...]
    for u in range(TB):
        ss = ss_ref[u]
        scale = ss[0, :]
        shift = ss[1, :]
        r_raw = r_ref[:, u].reshape(Bblk, N, H)
        x = r_raw * scale + shift
        x = _gat(x, amask, w1g_ref[...], a1s_ref[...], a1d_ref[...],
                 b1g_ref[...], Bblk, N, H)
        x = _gat(x, amask, w2g_ref[...], a2s_ref[...], a2d_ref[...],
                 b2g_ref[...], Bblk, N, H)
        h2, c2 = _lstm_step(x.reshape(Bblk * N, H), h_ref[...], c_ref[...],
                            wih4_ref, whh4_ref, b4_ref)
        h_ref[...] = h2
        c_ref[...] = c2
        h_out_ref[:, u] = jnp.concatenate(
            [r_raw, h2.reshape(Bblk, N, H)], axis=-1).reshape(Bblk, N, 2 * H)


def _gate_slabs(Wih, Whh, bih, bhh):
    H = Whh.shape[1]
    IN = Wih.shape[1]
    wih4 = Wih.reshape(4, H, IN).transpose(0, 2, 1)
    whh4 = Whh.reshape(4, H, H).transpose(0, 2, 1)
    b4 = (bih + bhh).reshape(4, 1, H)
    return wih4, whh4, b4


def kernel(feat, edge_index, params):
    p = params
    B, T, M, K, _ = feat.shape
    KR = K + 1
    MK = M * K
    N = M * KR
    EMB = p['phi_w2'].shape[0]
    H = p['Whh_s'].shape[1]
    Bblk = min(32, B)
    G = B // Bblk
    TB = 4 if T % 4 == 0 else 1
    TG = T // TB
    f32 = jnp.float32

    xx = feat[..., 0].reshape(B, T, 1, MK)
    yy = feat[..., 1].reshape(B, T, 1, MK)
    vv = feat[..., 2].reshape(B, T, 1, MK)
    w1t = p['phi_w1'].T
    b1 = p['phi_b1'][None, :]
    w2t = p['phi_w2'].T
    b2 = p['phi_b2'][None, :]
    wih4_s, whh4_s, b4_s = _gate_slabs(p['Wih_s'], p['Whh_s'],
                                       p['bih_s'], p['bhh_s'])
    wih4_t, whh4_t, b4_t = _gate_slabs(p['Wih_t'], p['Whh_t'],
                                       p['bih_t'], p['bhh_t'])
    w1g = p['g1_W'].T
    a1s = p['g1_asrc'][:, None]
    a1d = p['g1_adst'][:, None]
    b1g = p['g1_b'][None, :]
    w2g = p['g2_W'].T
    a2s = p['g2_asrc'][:, None]
    a2d = p['g2_adst'][:, None]
    b2g = p['g2_b'][None, :]

    leaf = jnp.arange(MK)
    S = (leaf[:, None] // K == jnp.arange(M)[None, :]).astype(f32)
    PL = (((leaf // K) * KR + leaf % K)[:, None]
          == jnp.arange(N)[None, :]).astype(f32)
    PR = (jnp.arange(M)[:, None] * KR + K
          == jnp.arange(N)[None, :]).astype(f32)

    src = edge_index[0]
    dst = edge_index[1]
    adj = jnp.zeros((N, N), f32).at[dst, src].add(1.0)
    amask = (1.0 - jnp.minimum(adj, 1.0)) * NEG_BIG

    const = lambda shape: pl.BlockSpec(shape, lambda i, t: tuple(0 for _ in shape))

    chan_spec = pl.BlockSpec((Bblk, TB, 1, MK), lambda i, t: (i, t, 0, 0))
    r, part = pl.pallas_call(
        functools.partial(_enc_body, Bblk, TB, MK, N, EMB, H),
        grid=(G, TG),
        in_specs=[
            chan_spec, chan_spec, chan_spec,
            const((MK, M)), const((MK, N)), const((M, N)),
            const((3, EMB)), const((1, EMB)),
            const((EMB, EMB)), const((1, EMB)),
            const((4, EMB, H)), const((4, H, H)), const((4, 1, H)),
        ],
        out_specs=[
            pl.BlockSpec((Bblk, TB, N, H), lambda i, t: (i, t, 0, 0)),
            pl.BlockSpec((1, TB, 2, H), lambda i, t: (i, t, 0, 0)),
        ],
        out_shape=[
            jax.ShapeDtypeStruct((B, T, N, H), f32),
            jax.ShapeDtypeStruct((G, T, 2, H), f32),
        ],
        scratch_shapes=[pltpu.VMEM((Bblk * N, H), f32),
                        pltpu.VMEM((Bblk * N, H), f32)],
        compiler_params=pltpu.CompilerParams(
            dimension_semantics=("parallel", "arbitrary")),
    )(xx, yy, vv, S, PL, PR, w1t, b1, w2t, b2, wih4_s, whh4_s, b4_s)

    ss = pl.pallas_call(
        functools.partial(_bn_body, float(B * N)),
        out_shape=jax.ShapeDtypeStruct((T, 2, H), f32),
    )(p['bn_gamma'][None, :], p['bn_beta'][None, :], part)

    h = pl.pallas_call(
        functools.partial(_dec_body, Bblk, TB, N, H),
        grid=(G, TG),
        in_specs=[
            pl.BlockSpec((Bblk, TB, N, H), lambda i, t: (i, t, 0, 0)),
            pl.BlockSpec((TB, 2, H), lambda i, t: (t, 0, 0)),
            const((N, N)),
            const((H, H)), const((H, 1)), const((H, 1)), const((1, H)),
            const((H, H)), const((H, 1)), const((H, 1)), const((1, H)),
            const((4, H, H)), const((4, H, H)), const((4, 1, H)),
        ],
        out_specs=pl.BlockSpec((Bblk, TB, N, 2 * H), lambda i, t: (i, t, 0, 0)),
        out_shape=jax.ShapeDtypeStruct((B, T, N, 2 * H), f32),
        scratch_shapes=[pltpu.VMEM((Bblk * N, H), f32),
                        pltpu.VMEM((Bblk * N, H), f32)],
        compiler_params=pltpu.CompilerParams(
            dimension_semantics=("parallel", "arbitrary")),
    )(r, ss, amask,
      w1g, a1s, a1d, b1g, w2g, a2s, a2d, b2g,
      wih4_t, whh4_t, b4_t)

    return (r, h)


# speedup vs baseline: 2.7410x; 1.0061x over previous
import functools

import jax
import jax.numpy as jnp
from jax.experimental import pallas as pl
from jax.experimental.pallas import tpu as pltpu

BN_EPS = 1e-5
NEG_BIG = -1e9


def _lstm_step(x, h, c, wih4_ref, whh4_ref, b4_ref):
    def gate(q):
        return (jnp.dot(x, wih4_ref[q], preferred_element_type=jnp.float32)
                + jnp.dot(h, whh4_ref[q], preferred_element_type=jnp.float32)
                + b4_ref[q])
    gi, gf, gg, go = gate(0), gate(1), gate(2), gate(3)
    c2 = jax.nn.sigmoid(gf) * c + jax.nn.sigmoid(gi) * jnp.tanh(gg)
    h2 = jax.nn.sigmoid(go) * jnp.tanh(c2)
    return h2, c2


def _enc_body(Bblk, TB, MK, N, EMB, H,
              xx_ref, yy_ref, vv_ref, S_ref, PL_ref, PR_ref,
              w1t_ref, b1_ref, w2t_ref, b2_ref,
              wih4_ref, whh4_ref, b4_ref,
              r_ref, part_ref, h_ref, c_ref):
    t = pl.program_id(1)

    @pl.when(t == 0)
    def _init():
        h_ref[...] = jnp.zeros_like(h_ref)
        c_ref[...] = jnp.zeros_like(c_ref)

    S = S_ref[...]
    PL = PL_ref[...]
    PR = PR_ref[...]
    for u in range(TB):
        xx = xx_ref[:, u].reshape(Bblk, MK)
        yy = yy_ref[:, u].reshape(Bblk, MK)
        vv = vv_ref[:, u].reshape(Bblk, MK)
        w = (vv > 0.5).astype(jnp.float32)
        sumw = jnp.dot(w, S, preferred_element_type=jnp.float32)
        denom = jnp.maximum(sumw, 1.0)
        rootx = jnp.dot(xx * w, S, preferred_element_type=jnp.float32) / denom
        rooty = jnp.dot(yy * w, S, preferred_element_type=jnp.float32) / denom
        rootv = (sumw > 0.5).astype(jnp.float32)

        def nodes(leaf, root):
            return (jnp.dot(leaf, PL, preferred_element_type=jnp.float32)
                    + jnp.dot(root, PR, preferred_element_type=jnp.float32))
        xN = nodes(xx, rootx)
        yN = nodes(yy, rooty)
        vN = nodes(vv, rootv)
        X = jnp.stack([xN, yN, vN], axis=-1).reshape(Bblk * N, 3)
        a1 = jax.nn.relu(jnp.dot(X, w1t_ref[...],
                                 preferred_element_type=jnp.float32) + b1_ref[...])
        v = (jnp.dot(a1, w2t_ref[...], preferred_element_type=jnp.float32)
             + b2_ref[...])

        h2, c2 = _lstm_step(v, h_ref[...], c_ref[...],
                            wih4_ref, whh4_ref, b4_ref)
        h_ref[...] = h2
        c_ref[...] = c2
        r_ref[:, u] = h2.reshape(Bblk, N, H)
        s = h2.sum(axis=0)[None, :]
        sq = (h2 * h2).sum(axis=0)[None, :]
        part_ref[:, u] = jnp.concatenate([s, sq], axis=0).reshape(1, 2, H)


def _bn_body(count, gamma_ref, beta_ref, part_ref, out_ref):
    p = part_ref[...].sum(axis=0)
    mean = p[:, 0, :] / count
    msq = p[:, 1, :] / count
    var = msq - mean * mean
    invstd = jax.lax.rsqrt(var + BN_EPS)
    scale = gamma_ref[...] * invstd
    shift = beta_ref[...] - mean * scale
    out_ref[...] = jnp.stack([scale, shift], axis=1)


def _gat(x, amask, wg, asr, ads, bg, Bblk, N, H):
    h1 = jnp.dot(x.reshape(Bblk * N, H), wg,
                 preferred_element_type=jnp.float32).reshape(Bblk, N, H)
    d3 = jax.lax.dot_general(h1, ads, (((2,), (0,)), ((), ())),
                             preferred_element_type=jnp.float32)
    s3 = jax.lax.dot_general(h1, asr, (((2,), (0,)), ((), ())),
                             preferred_element_type=jnp.float32)
    ones = jnp.ones_like(d3)
    A = jnp.concatenate([d3, ones], axis=-1)
    Bm = jnp.concatenate([ones, s3], axis=-1)
    e = jax.lax.dot_general(A, Bm, (((2,), (2,)), ((0,), (0,))),
                            preferred_element_type=jnp.float32)
    e = jnp.where(e >= 0, e, 0.2 * e) + amask
    emax = e.max(axis=-1, keepdims=True)
    ex = jnp.exp(e - emax)
    den = ex.sum(axis=-1, keepdims=True)
    alpha = ex / jnp.maximum(den, 1e-16)
    out = jax.lax.dot_general(
        alpha, h1, (((2,), (1,)), ((0,), (0,))),
        preferred_element_type=jnp.float32) + bg
    return jnp.where(out > 0, out, jnp.exp(jnp.minimum(out, 0.0)) - 1.0)


def _dec_body(Bblk, TB, N, H,
              r_ref, ss_ref, amask_ref,
              w1g_ref, a1s_ref, a1d_ref, b1g_ref,
              w2g_ref, a2s_ref, a2d_ref, b2g_ref,
              wih4_ref, whh4_ref, b4_ref,
              h_out_ref, h_ref, c_ref):
    t = pl.program_id(1)

    @pl.when(t == 0)
    def _init():
        h_ref[...] = jnp.zeros_like(h_ref)
        c_ref[...] = jnp.zeros_like(c_ref)

    amask = amask_ref[...]
    for u in range(TB):
        ss = ss_ref[u]
        scale = ss[0, :]
        shift = ss[1, :]
        r_raw = r_ref[:, u].reshape(Bblk, N, H)
        x = r_raw * scale + shift
        x = _gat(x, amask, w1g_ref[...], a1s_ref[...], a1d_ref[...],
                 b1g_ref[...], Bblk, N, H)
        x = _gat(x, amask, w2g_ref[...], a2s_ref[...], a2d_ref[...],
                 b2g_ref[...], Bblk, N, H)
        h2, c2 = _lstm_step(x.reshape(Bblk * N, H), h_ref[...], c_ref[...],
                            wih4_ref, whh4_ref, b4_ref)
        h_ref[...] = h2
        c_ref[...] = c2
        h_out_ref[:, u] = jnp.concatenate(
            [r_raw, h2.reshape(Bblk, N, H)], axis=-1).reshape(Bblk, N, 2 * H)


def _gate_slabs(Wih, Whh, bih, bhh):
    H = Whh.shape[1]
    IN = Wih.shape[1]
    wih4 = Wih.reshape(4, H, IN).transpose(0, 2, 1)
    whh4 = Whh.reshape(4, H, H).transpose(0, 2, 1)
    b4 = (bih + bhh).reshape(4, 1, H)
    return wih4, whh4, b4


def kernel(feat, edge_index, params):
    p = params
    B, T, M, K, _ = feat.shape
    KR = K + 1
    MK = M * K
    N = M * KR
    EMB = p['phi_w2'].shape[0]
    H = p['Whh_s'].shape[1]
    Bblk = min(32, B)
    G = B // Bblk
    TB = 8 if T % 8 == 0 else 1
    TG = T // TB
    f32 = jnp.float32

    xx = feat[..., 0].reshape(B, T, 1, MK)
    yy = feat[..., 1].reshape(B, T, 1, MK)
    vv = feat[..., 2].reshape(B, T, 1, MK)
    w1t = p['phi_w1'].T
    b1 = p['phi_b1'][None, :]
    w2t = p['phi_w2'].T
    b2 = p['phi_b2'][None, :]
    wih4_s, whh4_s, b4_s = _gate_slabs(p['Wih_s'], p['Whh_s'],
                                       p['bih_s'], p['bhh_s'])
    wih4_t, whh4_t, b4_t = _gate_slabs(p['Wih_t'], p['Whh_t'],
                                       p['bih_t'], p['bhh_t'])
    w1g = p['g1_W'].T
    a1s = p['g1_asrc'][:, None]
    a1d = p['g1_adst'][:, None]
    b1g = p['g1_b'][None, :]
    w2g = p['g2_W'].T
    a2s = p['g2_asrc'][:, None]
    a2d = p['g2_adst'][:, None]
    b2g = p['g2_b'][None, :]

    leaf = jnp.arange(MK)
    S = (leaf[:, None] // K == jnp.arange(M)[None, :]).astype(f32)
    PL = (((leaf // K) * KR + leaf % K)[:, None]
          == jnp.arange(N)[None, :]).astype(f32)
    PR = (jnp.arange(M)[:, None] * KR + K
          == jnp.arange(N)[None, :]).astype(f32)

    src = edge_index[0]
    dst = edge_index[1]
    adj = jnp.zeros((N, N), f32).at[dst, src].add(1.0)
    amask = (1.0 - jnp.minimum(adj, 1.0)) * NEG_BIG

    const = lambda shape: pl.BlockSpec(shape, lambda i, t: tuple(0 for _ in shape))

    chan_spec = pl.BlockSpec((Bblk, TB, 1, MK), lambda i, t: (i, t, 0, 0))
    r, part = pl.pallas_call(
        functools.partial(_enc_body, Bblk, TB, MK, N, EMB, H),
        grid=(G, TG),
        in_specs=[
            chan_spec, chan_spec, chan_spec,
            const((MK, M)), const((MK, N)), const((M, N)),
            const((3, EMB)), const((1, EMB)),
            const((EMB, EMB)), const((1, EMB)),
            const((4, EMB, H)), const((4, H, H)), const((4, 1, H)),
        ],
        out_specs=[
            pl.BlockSpec((Bblk, TB, N, H), lambda i, t: (i, t, 0, 0)),
            pl.BlockSpec((1, TB, 2, H), lambda i, t: (i, t, 0, 0)),
        ],
        out_shape=[
            jax.ShapeDtypeStruct((B, T, N, H), f32),
            jax.ShapeDtypeStruct((G, T, 2, H), f32),
        ],
        scratch_shapes=[pltpu.VMEM((Bblk * N, H), f32),
                        pltpu.VMEM((Bblk * N, H), f32)],
        compiler_params=pltpu.CompilerParams(
            dimension_semantics=("parallel", "arbitrary")),
    )(xx, yy, vv, S, PL, PR, w1t, b1, w2t, b2, wih4_s, whh4_s, b4_s)

    ss = pl.pallas_call(
        functools.partial(_bn_body, float(B * N)),
        out_shape=jax.ShapeDtypeStruct((T, 2, H), f32),
    )(p['bn_gamma'][None, :], p['bn_beta'][None, :], part)

    h = pl.pallas_call(
        functools.partial(_dec_body, Bblk, TB, N, H),
        grid=(G, TG),
        in_specs=[
            pl.BlockSpec((Bblk, TB, N, H), lambda i, t: (i, t, 0, 0)),
            pl.BlockSpec((TB, 2, H), lambda i, t: (t, 0, 0)),
            const((N, N)),
            const((H, H)), const((H, 1)), const((H, 1)), const((1, H)),
            const((H, H)), const((H, 1)), const((H, 1)), const((1, H)),
            const((4, H, H)), const((4, H, H)), const((4, 1, H)),
        ],
        out_specs=pl.BlockSpec((Bblk, TB, N, 2 * H), lambda i, t: (i, t, 0, 0)),
        out_shape=jax.ShapeDtypeStruct((B, T, N, 2 * H), f32),
        scratch_shapes=[pltpu.VMEM((Bblk * N, H), f32),
                        pltpu.VMEM((Bblk * N, H), f32)],
        compiler_params=pltpu.CompilerParams(
            dimension_semantics=("parallel", "arbitrary")),
    )(r, ss, amask,
      w1g, a1s, a1d, b1g, w2g, a2s, a2d, b2g,
      wih4_t, whh4_t, b4_t)

    return (r, h)
